# baseline (speedup 1.0000x reference)
import jax
import jax.numpy as jnp
from jax import lax
from jax.experimental import pallas as pl
from jax.experimental.pallas import tpu as pltpu
from jax.experimental.pallas import tpu_sc as plsc

NUM_CAMERAS = 10000
NUM_RAYS = 65536
CAM_PAD = 10240
D_OUT = 12
D_PAD = 16

_NC = 2
_NS = 16
_NW = _NC * _NS
_CHUNK = 128
_ROWS_PER_W = NUM_RAYS // _NW
_CHUNKS_PER_W = _ROWS_PER_W // _CHUNK
_CAMS_PER_TILE = CAM_PAD // _NS
_L = 16
_OUT_ROWS = NUM_RAYS * D_OUT // 128
_OUT_ROWS_W = _OUT_ROWS // _NW

_F1 = [-1.0 / 1307674368000.0, 1.0 / 6227020800.0, -1.0 / 39916800.0,
       1.0 / 362880.0, -1.0 / 5040.0, 1.0 / 120.0, -1.0 / 6.0, 1.0]
_F2 = [-1.0 / 20922789888000.0, 1.0 / 87178291200.0, -1.0 / 479001600.0,
       1.0 / 3628800.0, -1.0 / 40320.0, 1.0 / 720.0, -1.0 / 24.0, 0.5]


def _horner(coeffs, s):
    acc = jnp.full((_L,), coeffs[0], jnp.float32)
    for c in coeffs[1:]:
        acc = acc * s + c
    return acc


def _splat(v):
    return jnp.full((_L,), v, jnp.int32)


def _sc_body(pose_hbm, idx_hbm, out_hbm, table_sc,
             pose_v, table_v, idx_v, rows_v, out_v, sem, sem2):
    sid = lax.axis_index("s")
    cid = lax.axis_index("c")
    wid = sid * _NC + cid
    iota = lax.iota(jnp.int32, _L)

    idx_cp = pltpu.async_copy(
        idx_hbm.at[pl.ds(wid * _CHUNKS_PER_W, _CHUNKS_PER_W)], idx_v, sem2)

    pltpu.sync_copy(pose_hbm.at[pl.ds(sid * _CAMS_PER_TILE, _CAMS_PER_TILE)],
                    pose_v)

    def build(g, carry):
        lid = g * _L + iota
        t0 = plsc.load_gather(pose_v, [lid, _splat(0)])
        t1 = plsc.load_gather(pose_v, [lid, _splat(1)])
        t2 = plsc.load_gather(pose_v, [lid, _splat(2)])
        w0 = plsc.load_gather(pose_v, [lid, _splat(3)])
        w1 = plsc.load_gather(pose_v, [lid, _splat(4)])
        w2 = plsc.load_gather(pose_v, [lid, _splat(5)])
        nrm = w0 * w0 + w1 * w1 + w2 * w2
        s = jnp.maximum(nrm, 1e-4)
        fac1 = _horner(_F1, s)
        fac2 = _horner(_F2, s)
        f01 = fac2 * (w0 * w1)
        f02 = fac2 * (w0 * w2)
        f12 = fac2 * (w1 * w2)
        vals = (fac2 * (w0 * w0 - nrm) + 1.0,
                f01 - fac1 * w2,
                f02 + fac1 * w1,
                t0,
                f01 + fac1 * w2,
                fac2 * (w1 * w1 - nrm) + 1.0,
                f12 - fac1 * w0,
                t1,
                f02 - fac1 * w1,
                f12 + fac1 * w0,
                fac2 * (w2 * w2 - nrm) + 1.0,
                t2)
        for c, v in enumerate(vals):
            plsc.store_scatter(table_v, [lid, _splat(c)], v)
        return carry

    lax.fori_loop(0, _CAMS_PER_TILE // _L, build, 0)
    pltpu.sync_copy(table_v,
                    table_sc.at[cid, pl.ds(sid * _CAMS_PER_TILE,
                                           _CAMS_PER_TILE)])
    plsc.subcore_barrier()

    idx_cp.wait()
    tab = table_sc.at[cid]
    copies = [pltpu.async_copy(tab.at[idx_v.at[j]], rows_v.at[j], sem)
              for j in range(_CHUNKS_PER_W)]

    iota12 = iota * D_OUT
    offs = []
    for p in (0, 1):
        per_c = []
        for c in range(D_OUT):
            off = iota12 + (p * 64 + c)
            per_c.append((lax.shift_right_logical(off, 7),
                          jnp.bitwise_and(off, 127)))
        offs.append(per_c)
    for c in copies:
        c.wait()
    for j in range(_CHUNKS_PER_W):

        def repack(k, carry, j=j):
            for p in (0, 1):
                rays = k * 32 + (p * _L + iota)
                brow = j * (_CHUNK * D_OUT // 128) + k * 3 + p
                for c in range(D_OUT):
                    ro, co = offs[p][c]
                    v = plsc.load_gather(rows_v, [_splat(j), rays, _splat(c)])
                    plsc.store_scatter(out_v, [brow + ro, co], v)
            return carry

        lax.fori_loop(0, _CHUNK // _L // 2, repack, 0)

    pltpu.sync_copy(out_v, out_hbm.at[pl.ds(wid * _OUT_ROWS_W, _OUT_ROWS_W)])


def _run_sc(pose_pad, idx2d):
    mesh = plsc.VectorSubcoreMesh(core_axis_name="c", subcore_axis_name="s")
    fn = pl.kernel(
        _sc_body,
        out_type=jax.ShapeDtypeStruct((_OUT_ROWS, 128), jnp.float32),
        mesh=mesh,
        scratch_types=[
            pltpu.HBM((_NC, CAM_PAD, D_PAD), jnp.float32),
            pltpu.VMEM((_CAMS_PER_TILE, 6), jnp.float32),
            pltpu.VMEM((_CAMS_PER_TILE, D_PAD), jnp.float32),
            pltpu.VMEM((_CHUNKS_PER_W, _CHUNK), jnp.int32),
            pltpu.VMEM((_CHUNKS_PER_W, _CHUNK, D_PAD), jnp.float32),
            pltpu.VMEM((_OUT_ROWS_W, 128), jnp.float32),
            pltpu.SemaphoreType.DMA,
            pltpu.SemaphoreType.DMA,
        ],
        compiler_params=pltpu.CompilerParams(use_tc_tiling_on_sc=False,
                                             needs_layout_passes=False),
    )
    return fn(pose_pad, idx2d)


def kernel(camera_indices, pose_adjustment):
    pose_pad = jnp.pad(pose_adjustment, ((0, CAM_PAD - NUM_CAMERAS), (0, 0)))
    idx2d = camera_indices[:, 0].reshape(NUM_RAYS // _CHUNK, _CHUNK)
    packed = _run_sc(pose_pad, idx2d)
    return packed.reshape(NUM_RAYS, 3, 4)


# speedup vs baseline: 3.4072x; 3.4072x over previous
import jax
import jax.numpy as jnp
from jax import lax
from jax.experimental import pallas as pl
from jax.experimental.pallas import tpu as pltpu
from jax.experimental.pallas import tpu_sc as plsc

NUM_CAMERAS = 10000
NUM_RAYS = 65536
CAM_PAD = 10240
D_OUT = 12
D_PAD = 16

_NC = 2
_NS = 16
_NW = _NC * _NS
_CHUNK = 128
_ROWS_PER_W = NUM_RAYS // _NW
_CHUNKS_PER_W = _ROWS_PER_W // _CHUNK


def _expmap_table_body(p_ref, o_ref):
    t0, t1, t2 = p_ref[0], p_ref[1], p_ref[2]
    w0, w1, w2 = p_ref[3], p_ref[4], p_ref[5]
    nrm = w0 * w0 + w1 * w1 + w2 * w2
    s = jnp.maximum(nrm, 1e-4)
    a = jnp.sqrt(s)
    fac1 = jnp.sin(a) / a
    fac2 = (1.0 - jnp.cos(a)) / s
    f2_01 = fac2 * (w0 * w1)
    f2_02 = fac2 * (w0 * w2)
    f2_12 = fac2 * (w1 * w2)
    o_ref[0] = fac2 * (w0 * w0 - nrm) + 1.0
    o_ref[1] = f2_01 - fac1 * w2
    o_ref[2] = f2_02 + fac1 * w1
    o_ref[3] = t0
    o_ref[4] = f2_01 + fac1 * w2
    o_ref[5] = fac2 * (w1 * w1 - nrm) + 1.0
    o_ref[6] = f2_12 - fac1 * w0
    o_ref[7] = t1
    o_ref[8] = f2_02 - fac1 * w1
    o_ref[9] = f2_12 + fac1 * w0
    o_ref[10] = fac2 * (w2 * w2 - nrm) + 1.0
    o_ref[11] = t2
    zero = jnp.zeros_like(t0)
    o_ref[12] = zero
    o_ref[13] = zero
    o_ref[14] = zero
    o_ref[15] = zero


def _build_table(params_t):
    return pl.pallas_call(
        _expmap_table_body,
        out_shape=jax.ShapeDtypeStruct((D_PAD, 80, 128), jnp.float32),
    )(params_t)


def _gather_body(table_hbm, idx_hbm, out_hbm, idx_v, rows_v, sem):
    wid = lax.axis_index("s") * _NC + lax.axis_index("c")
    base = wid * _CHUNKS_PER_W
    pltpu.sync_copy(idx_hbm.at[pl.ds(base, _CHUNKS_PER_W)], idx_v)
    copies = []
    for j in range(_CHUNKS_PER_W):
        copies.append(
            pltpu.async_copy(table_hbm.at[idx_v.at[j]], rows_v.at[j], sem))
    for c in copies:
        c.wait()
    pltpu.sync_copy(rows_v, out_hbm.at[pl.ds(base, _CHUNKS_PER_W)])


def _gather_rows(table, idx2d):
    mesh = plsc.VectorSubcoreMesh(core_axis_name="c", subcore_axis_name="s")
    grab = pl.kernel(
        _gather_body,
        out_type=jax.ShapeDtypeStruct((NUM_RAYS // _CHUNK, _CHUNK, D_PAD),
                                      jnp.float32),
        mesh=mesh,
        scratch_types=[
            pltpu.VMEM((_CHUNKS_PER_W, _CHUNK), jnp.int32),
            pltpu.VMEM((_CHUNKS_PER_W, _CHUNK, D_PAD), jnp.float32),
            pltpu.SemaphoreType.DMA,
        ],
        compiler_params=pltpu.CompilerParams(use_tc_tiling_on_sc=False),
    )
    return grab(table, idx2d)


def kernel(camera_indices, pose_adjustment):
    params_t = jnp.transpose(pose_adjustment)
    params_t = jnp.pad(params_t, ((0, 0), (0, CAM_PAD - NUM_CAMERAS)))
    params_t = params_t.reshape(6, CAM_PAD // 128, 128)
    table_t = _build_table(params_t)
    table = jnp.transpose(table_t.reshape(D_PAD, CAM_PAD))
    idx2d = camera_indices[:, 0].reshape(NUM_RAYS // _CHUNK, _CHUNK)
    rows = _gather_rows(table, idx2d)
    return rows[:, :, :D_OUT].reshape(NUM_RAYS, 3, 4)
